# Initial kernel scaffold; baseline (speedup 1.0000x reference)
import functools

import jax
import jax.numpy as jnp
from jax import lax
from jax.experimental import pallas as pl
from jax.experimental.pallas import tpu as pltpu
from jax.experimental.pallas import tpu_sc as plsc

_NC = 2
_NS = 16
_NW = _NC * _NS
_LANES = 16


def _leaky(x):
    return jnp.maximum(x, 0.1 * x)


def _sc1(tab0, idx32, wp_f1, M):
    P = M // _NW
    NB = 4
    nblk = P // NB

    mesh = plsc.VectorSubcoreMesh(core_axis_name="c", subcore_axis_name="s")

    @functools.partial(
        pl.kernel,
        mesh=mesh,
        out_type=(
            jax.ShapeDtypeStruct((M, 3, 16), jnp.float32),
            jax.ShapeDtypeStruct((M, 8), jnp.float32),
        ),
        scratch_types=[
            pltpu.VMEM((128,), jnp.int32),
            pltpu.VMEM((128, 16), jnp.float32),
            pltpu.VMEM((NB, 16), jnp.float32),
            pltpu.VMEM((NB, 3, 16), jnp.float32),
            pltpu.VMEM((NB, 8), jnp.float32),
            pltpu.VMEM((4, 4), jnp.float32),
            pltpu.SemaphoreType.DMA,
        ],
    )
    def k(tab_hbm, idx_hbm, w_hbm, rel_hbm, agg_hbm,
          idx_v, rows_v, cen_v, rel_v, out_v, w_v, sem):
        wid = lax.axis_index("s") * _NC + lax.axis_index("c")
        pltpu.sync_copy(w_hbm, w_v)
        lane = lax.iota(jnp.int32, (16,))

        def block(t, _):
            p0 = wid * P + t * NB
            pltpu.sync_copy(idx_hbm.at[pl.ds(p0 * 32, 128)], idx_v)
            pltpu.async_copy(tab_hbm.at[idx_v], rows_v, sem).wait()
            pltpu.sync_copy(tab_hbm.at[pl.ds(p0, NB)], cen_v)

            def point(i, _):
                acc = [jnp.float32(0.0)] * 3
                for h in range(2):
                    ridx = i * 32 + h * 16 + lane
                    comps = []
                    for d in range(3):
                        g = plsc.load_gather(
                            rows_v, [ridx, jnp.full((16,), d, jnp.int32)])
                        r = g - cen_v[i, d]
                        comps.append(r)
                        if h == 0:
                            rel_v[i, d, :] = r
                    fl = [
                        plsc.load_gather(
                            rows_v, [ridx, jnp.full((16,), 3 + d, jnp.int32)])
                        for d in range(3)
                    ]
                    for c in range(3):
                        t0 = (comps[0] * w_v[0, c] + comps[1] * w_v[1, c]
                              + comps[2] * w_v[2, c] + w_v[3, c])
                        w = _leaky(t0)
                        acc[c] = acc[c] + jnp.sum(w * fl[c])
                for c in range(3):
                    out_v[i, c] = acc[c]
                for c in range(3, 8):
                    out_v[i, c] = jnp.float32(0.0)
                return 0

            lax.fori_loop(0, NB, point, 0)
            pltpu.sync_copy(rel_v, rel_hbm.at[pl.ds(p0, NB)])
            pltpu.sync_copy(out_v, agg_hbm.at[pl.ds(p0, NB)])
            return 0

        lax.fori_loop(0, nblk, block, 0)

    return k(tab0, idx32, wp_f1)


def _sc_layer(table, idx16, rel16, wp, M, C):
    P = M // _NW
    NB = 8
    nblk = P // NB
    nc = C // _LANES

    mesh = plsc.VectorSubcoreMesh(core_axis_name="c", subcore_axis_name="s")

    @functools.partial(
        pl.kernel,
        mesh=mesh,
        out_type=jax.ShapeDtypeStruct((M, C), jnp.float32),
        scratch_types=[
            pltpu.VMEM((128,), jnp.int32),
            pltpu.VMEM((128, C), jnp.float32),
            pltpu.VMEM((NB, 3, 16), jnp.float32),
            pltpu.VMEM((NB, C), jnp.float32),
            pltpu.VMEM((4, C), jnp.float32),
            pltpu.SemaphoreType.DMA,
        ],
    )
    def k(tab_hbm, idx_hbm, rel_hbm, w_hbm, agg_hbm,
          idx_v, rows_v, rel_v, out_v, w_v, sem):
        wid = lax.axis_index("s") * _NC + lax.axis_index("c")
        pltpu.sync_copy(w_hbm, w_v)
        w0 = [w_v[0, pl.ds(c * 16, 16)] for c in range(nc)]
        w1 = [w_v[1, pl.ds(c * 16, 16)] for c in range(nc)]
        w2 = [w_v[2, pl.ds(c * 16, 16)] for c in range(nc)]
        wb = [w_v[3, pl.ds(c * 16, 16)] for c in range(nc)]

        def block(t, _):
            p0 = wid * P + t * NB
            pltpu.sync_copy(idx_hbm.at[pl.ds(p0 * 16, 128)], idx_v)
            pltpu.async_copy(tab_hbm.at[idx_v], rows_v, sem).wait()
            pltpu.sync_copy(rel_hbm.at[pl.ds(p0, NB)], rel_v)

            def point(i, _):
                accs = [jnp.zeros((16,), jnp.float32) for _ in range(nc)]
                for j in range(16):
                    rx = rel_v[i, 0, j]
                    ry = rel_v[i, 1, j]
                    rz = rel_v[i, 2, j]
                    row = i * 16 + j
                    for c in range(nc):
                        t0 = rx * w0[c] + ry * w1[c] + rz * w2[c] + wb[c]
                        w = _leaky(t0)
                        f = rows_v[row, pl.ds(c * 16, 16)]
                        accs[c] = accs[c] + w * f
                for c in range(nc):
                    out_v[i, pl.ds(c * 16, 16)] = accs[c]
                return 0

            lax.fori_loop(0, NB, point, 0)
            pltpu.sync_copy(out_v, agg_hbm.at[pl.ds(p0, NB)])
            return 0

        lax.fori_loop(0, nblk, block, 0)

    return k(table, idx16, rel16, wp)


def _tc_linear(x, wT, b):
    M, C = x.shape
    O = wT.shape[1]
    BM = 2048

    def body(x_ref, w_ref, b_ref, o_ref):
        y = jnp.dot(x_ref[...], w_ref[...],
                    preferred_element_type=jnp.float32,
                    precision=lax.Precision.HIGHEST)
        y = y + b_ref[...]
        o_ref[...] = jnp.maximum(y, 0.1 * y)

    return pl.pallas_call(
        body,
        grid=(M // BM,),
        in_specs=[
            pl.BlockSpec((BM, C), lambda i: (i, 0)),
            pl.BlockSpec((C, O), lambda i: (0, 0)),
            pl.BlockSpec((1, O), lambda i: (0, 0)),
        ],
        out_specs=pl.BlockSpec((BM, O), lambda i: (i, 0)),
        out_shape=jax.ShapeDtypeStruct((M, O), jnp.float32),
    )(x, wT, b)


def _wpack(Wwn, bwn, pad_to=None):
    wp = jnp.concatenate([Wwn.T, bwn[None, :]], axis=0)
    if pad_to is not None and wp.shape[1] < pad_to:
        wp = jnp.pad(wp, ((0, 0), (0, pad_to - wp.shape[1])))
    return wp.astype(jnp.float32)


def _linpack(Wlin, blin, cin_pad=None, o_pad=128):
    wT = Wlin.T
    C, O = wT.shape
    if cin_pad is not None and C < cin_pad:
        wT = jnp.pad(wT, ((0, cin_pad - C), (0, 0)))
    if O < o_pad:
        wT = jnp.pad(wT, ((0, 0), (0, o_pad - O)))
        blin = jnp.pad(blin, (0, o_pad - O))
    return wT.astype(jnp.float32), blin[None, :].astype(jnp.float32)


def kernel(xyz, flow, corr, knn_indices,
           Wwn_c1, bwn_c1, Wlin_c1, blin_c1,
           Wwn_f1, bwn_f1, Wlin_f1, blin_f1,
           Wwn_f2, bwn_f2, Wlin_f2, blin_f2,
           Wwn_o, bwn_o, Wlin_o, blin_o):
    B, _, N = xyz.shape
    M = B * N
    f32 = jnp.float32
    corr = corr.astype(f32)
    flow = flow.astype(f32)

    xyzT = jnp.swapaxes(xyz, 1, 2).reshape(M, 3)
    flowT = jnp.swapaxes(flow, 1, 2).reshape(M, 3)
    tab0 = jnp.concatenate(
        [xyzT, flowT, jnp.zeros((M, 10), f32)], axis=1)
    offs = (jnp.arange(B, dtype=jnp.int32) * N)[:, None, None]
    idxg = knn_indices.astype(jnp.int32) + offs
    idx32 = idxg.reshape(M * 32)
    idx16 = idxg[:, :, :16].reshape(M * 16)
    corrT = jnp.swapaxes(corr, 1, 2).reshape(M, 128)

    rel16, agg_f1 = _sc1(tab0, idx32, _wpack(Wwn_f1, bwn_f1, pad_to=4), M)
    wT, bb = _linpack(Wlin_f1, blin_f1, cin_pad=8)
    ff1 = _tc_linear(agg_f1, wT, bb)[:, :32]

    agg_c1 = _sc_layer(corrT, idx16, rel16, _wpack(Wwn_c1, bwn_c1), M, 128)
    wT, bb = _linpack(Wlin_c1, blin_c1)
    cf = _tc_linear(agg_c1, wT, bb)

    agg_f2 = _sc_layer(ff1, idx16, rel16, _wpack(Wwn_f2, bwn_f2), M, 32)
    wT, bb = _linpack(Wlin_f2, blin_f2)
    ff2 = _tc_linear(agg_f2, wT, bb)[:, :16]

    tab_o = jnp.concatenate([cf, ff2], axis=1)
    agg_o = _sc_layer(tab_o, idx16, rel16, _wpack(Wwn_o, bwn_o), M, 144)
    wT, bb = _linpack(Wlin_o, blin_o)
    out = _tc_linear(agg_o, wT, bb)[:, :125]

    out = jnp.swapaxes(out.reshape(B, N, 125), 1, 2)
    return jnp.concatenate(
        [out, flow], axis=1)


# speedup vs baseline: 29.5946x; 29.5946x over previous
import functools

import jax
import jax.numpy as jnp
from jax import lax
from jax.experimental import pallas as pl
from jax.experimental.pallas import tpu as pltpu
from jax.experimental.pallas import tpu_sc as plsc

_NC = 2
_NS = 16
_NW = _NC * _NS
_LANES = 16


def _leaky(x):
    return jnp.maximum(x, 0.1 * x)


def _sc1(tab0, idx32, wp_f1, M):
    P = M // _NW
    NB = 4
    nblk = P // NB

    mesh = plsc.VectorSubcoreMesh(core_axis_name="c", subcore_axis_name="s")

    @functools.partial(
        pl.kernel,
        mesh=mesh,
        compiler_params=pltpu.CompilerParams(use_tc_tiling_on_sc=False),
        out_type=jax.ShapeDtypeStruct((M, 16), jnp.float32),
        scratch_types=[
            pltpu.VMEM((128,), jnp.int32),
            pltpu.VMEM((128, 32), jnp.float32),
            pltpu.VMEM((NB, 16), jnp.float32),
            pltpu.VMEM((NB, 16), jnp.float32),
            pltpu.VMEM((4, 16), jnp.float32),
            pltpu.SemaphoreType.DMA,
        ],
    )
    def k(tab_hbm, idx_hbm, w_hbm, agg_hbm,
          idx_v, rows_v, cen_v, out_v, w_v, sem):
        wid = lax.axis_index("s") * _NC + lax.axis_index("c")
        pltpu.sync_copy(w_hbm, w_v)
        w0 = w_v[0, :]
        w1 = w_v[1, :]
        w2 = w_v[2, :]
        wb = w_v[3, :]

        def block(t, _):
            p0 = wid * P + t * NB
            pltpu.sync_copy(idx_hbm.at[pl.ds(p0 * 32, 128)], idx_v)
            pltpu.async_copy(tab_hbm.at[idx_v], rows_v, sem).wait()
            pltpu.sync_copy(tab_hbm.at[pl.ds(p0, NB), pl.ds(0, 16)], cen_v)

            def point(i, _):
                cen = cen_v[i, :]
                acc = jnp.zeros((16,), jnp.float32)
                for j in range(32):
                    row = i * 32 + j
                    xyzv = rows_v[row, pl.ds(0, 16)]
                    flv = rows_v[row, pl.ds(16, 16)]
                    diff = xyzv - cen
                    rx = diff[0]
                    ry = diff[1]
                    rz = diff[2]
                    w = _leaky(rx * w0 + ry * w1 + rz * w2 + wb)
                    acc = acc + w * flv
                out_v[i, :] = acc
                return 0

            lax.fori_loop(0, NB, point, 0)
            pltpu.sync_copy(out_v, agg_hbm.at[pl.ds(p0, NB)])
            return 0

        lax.fori_loop(0, nblk, block, 0)

    return k(tab0, idx32, wp_f1)


def _sc_layer(table, idx16, wp, M, C):
    P = M // _NW
    NB = 8
    nblk = P // NB
    nc = C // _LANES
    R = C + 16

    mesh = plsc.VectorSubcoreMesh(core_axis_name="c", subcore_axis_name="s")

    @functools.partial(
        pl.kernel,
        mesh=mesh,
        compiler_params=pltpu.CompilerParams(use_tc_tiling_on_sc=False),
        out_type=jax.ShapeDtypeStruct((M, C), jnp.float32),
        scratch_types=[
            pltpu.VMEM((128,), jnp.int32),
            pltpu.VMEM((128, R), jnp.float32),
            pltpu.VMEM((NB, 16), jnp.float32),
            pltpu.VMEM((NB, C), jnp.float32),
            pltpu.VMEM((4, C), jnp.float32),
            pltpu.SemaphoreType.DMA,
        ],
    )
    def k(tab_hbm, idx_hbm, w_hbm, agg_hbm,
          idx_v, rows_v, cen_v, out_v, w_v, sem):
        wid = lax.axis_index("s") * _NC + lax.axis_index("c")
        pltpu.sync_copy(w_hbm, w_v)
        w0 = [w_v[0, pl.ds(c * 16, 16)] for c in range(nc)]
        w1 = [w_v[1, pl.ds(c * 16, 16)] for c in range(nc)]
        w2 = [w_v[2, pl.ds(c * 16, 16)] for c in range(nc)]
        wb = [w_v[3, pl.ds(c * 16, 16)] for c in range(nc)]

        def block(t, _):
            p0 = wid * P + t * NB
            pltpu.sync_copy(idx_hbm.at[pl.ds(p0 * 16, 128)], idx_v)
            pltpu.async_copy(tab_hbm.at[idx_v], rows_v, sem).wait()
            pltpu.sync_copy(tab_hbm.at[pl.ds(p0, NB), pl.ds(C, 16)], cen_v)

            def point(i, _):
                cen = cen_v[i, :]
                accs = [jnp.zeros((16,), jnp.float32) for _ in range(nc)]
                for j in range(16):
                    row = i * 16 + j
                    diff = rows_v[row, pl.ds(C, 16)] - cen
                    rx = diff[0]
                    ry = diff[1]
                    rz = diff[2]
                    for c in range(nc):
                        t0 = rx * w0[c] + ry * w1[c] + rz * w2[c] + wb[c]
                        w = _leaky(t0)
                        f = rows_v[row, pl.ds(c * 16, 16)]
                        accs[c] = accs[c] + w * f
                for c in range(nc):
                    out_v[i, pl.ds(c * 16, 16)] = accs[c]
                return 0

            lax.fori_loop(0, NB, point, 0)
            pltpu.sync_copy(out_v, agg_hbm.at[pl.ds(p0, NB)])
            return 0

        lax.fori_loop(0, nblk, block, 0)

    return k(table, idx16, wp)


def _tc_linear(x, wT, b):
    M, C = x.shape
    O = wT.shape[1]
    BM = 2048

    def body(x_ref, w_ref, b_ref, o_ref):
        y = jnp.dot(x_ref[...], w_ref[...],
                    preferred_element_type=jnp.float32,
                    precision=lax.Precision.HIGHEST)
        y = y + b_ref[...]
        o_ref[...] = jnp.maximum(y, 0.1 * y)

    return pl.pallas_call(
        body,
        grid=(M // BM,),
        in_specs=[
            pl.BlockSpec((BM, C), lambda i: (i, 0)),
            pl.BlockSpec((C, O), lambda i: (0, 0)),
            pl.BlockSpec((1, O), lambda i: (0, 0)),
        ],
        out_specs=pl.BlockSpec((BM, O), lambda i: (i, 0)),
        out_shape=jax.ShapeDtypeStruct((M, O), jnp.float32),
    )(x, wT, b)


def _wpack(Wwn, bwn, pad_to=None):
    wp = jnp.concatenate([Wwn.T, bwn[None, :]], axis=0)
    if pad_to is not None and wp.shape[1] < pad_to:
        wp = jnp.pad(wp, ((0, 0), (0, pad_to - wp.shape[1])))
    return wp.astype(jnp.float32)


def _linpack(Wlin, blin, cin_pad=None, o_pad=128):
    wT = Wlin.T
    C, O = wT.shape
    if cin_pad is not None and C < cin_pad:
        wT = jnp.pad(wT, ((0, cin_pad - C), (0, 0)))
    if O < o_pad:
        wT = jnp.pad(wT, ((0, 0), (0, o_pad - O)))
        blin = jnp.pad(blin, (0, o_pad - O))
    return wT.astype(jnp.float32), blin[None, :].astype(jnp.float32)


def kernel(xyz, flow, corr, knn_indices,
           Wwn_c1, bwn_c1, Wlin_c1, blin_c1,
           Wwn_f1, bwn_f1, Wlin_f1, blin_f1,
           Wwn_f2, bwn_f2, Wlin_f2, blin_f2,
           Wwn_o, bwn_o, Wlin_o, blin_o):
    B, _, N = xyz.shape
    M = B * N
    f32 = jnp.float32
    corr = corr.astype(f32)
    flow = flow.astype(f32)

    xyzT = jnp.swapaxes(xyz, 1, 2).reshape(M, 3)
    flowT = jnp.swapaxes(flow, 1, 2).reshape(M, 3)
    z13 = jnp.zeros((M, 13), f32)
    xyz16 = jnp.concatenate([xyzT, z13], axis=1)
    tab0 = jnp.concatenate(
        [xyz16, flowT, z13], axis=1)
    offs = (jnp.arange(B, dtype=jnp.int32) * N)[:, None, None]
    idxg = knn_indices.astype(jnp.int32) + offs
    idx32 = idxg.reshape(M * 32)
    idx16 = idxg[:, :, :16].reshape(M * 16)
    corrT = jnp.swapaxes(corr, 1, 2).reshape(M, 128)

    agg_f1 = _sc1(tab0, idx32, _wpack(Wwn_f1, bwn_f1, pad_to=16), M)
    wT, bb = _linpack(Wlin_f1, blin_f1, cin_pad=16)
    ff1 = _tc_linear(agg_f1, wT, bb)[:, :32]

    tab_c = jnp.concatenate([corrT, xyz16], axis=1)
    agg_c1 = _sc_layer(tab_c, idx16, _wpack(Wwn_c1, bwn_c1), M, 128)
    wT, bb = _linpack(Wlin_c1, blin_c1)
    cf = _tc_linear(agg_c1, wT, bb)

    tab_f = jnp.concatenate([ff1, xyz16], axis=1)
    agg_f2 = _sc_layer(tab_f, idx16, _wpack(Wwn_f2, bwn_f2), M, 32)
    wT, bb = _linpack(Wlin_f2, blin_f2)
    ff2 = _tc_linear(agg_f2, wT, bb)[:, :16]

    tab_o = jnp.concatenate([cf, ff2, xyz16], axis=1)
    agg_o = _sc_layer(tab_o, idx16, _wpack(Wwn_o, bwn_o), M, 144)
    wT, bb = _linpack(Wlin_o, blin_o)
    out = _tc_linear(agg_o, wT, bb)[:, :125]

    out = jnp.swapaxes(out.reshape(B, N, 125), 1, 2)
    return jnp.concatenate(
        [out, flow], axis=1)


# speedup vs baseline: 62.3089x; 2.1054x over previous
import functools

import jax
import jax.numpy as jnp
from jax import lax
from jax.experimental import pallas as pl
from jax.experimental.pallas import tpu as pltpu
from jax.experimental.pallas import tpu_sc as plsc

_NC = 2
_NS = 16
_NW = _NC * _NS
_LANES = 16


def _leaky(x):
    return jnp.maximum(x, 0.1 * x)


def _sc_layer(table, idx, wp, M, C, K, NB, feat_off, R, Co):
    P = M // _NW
    RB = NB * K
    nh = RB // 128
    assert RB % 128 == 0
    nblk = P // NB
    assert nblk % 2 == 0
    nc = Co // _LANES
    xyz_off = C if feat_off == 0 else 0

    mesh = plsc.VectorSubcoreMesh(core_axis_name="c", subcore_axis_name="s")

    @functools.partial(
        pl.kernel,
        mesh=mesh,
        compiler_params=pltpu.CompilerParams(use_tc_tiling_on_sc=False),
        out_type=jax.ShapeDtypeStruct((M, Co), jnp.float32),
        scratch_types=[
            pltpu.VMEM((P * K,), jnp.int32),
            pltpu.VMEM((P, 16), jnp.float32),
            pltpu.VMEM((2, RB, R), jnp.float32),
            pltpu.VMEM((2, NB, Co), jnp.float32),
            pltpu.VMEM((4, Co), jnp.float32),
            pltpu.SemaphoreType.DMA,
            pltpu.SemaphoreType.DMA,
            pltpu.SemaphoreType.DMA,
            pltpu.SemaphoreType.DMA,
        ],
    )
    def k(tab_hbm, idx_hbm, w_hbm, agg_hbm,
          idx_v, cen_v, rows_v, out_v, w_v, sg0, sg1, so0, so1):
        wid = lax.axis_index("s") * _NC + lax.axis_index("c")
        base = wid * P
        pltpu.sync_copy(w_hbm, w_v)
        pltpu.sync_copy(idx_hbm.at[pl.ds(base * K, P * K)], idx_v)
        pltpu.sync_copy(
            tab_hbm.at[pl.ds(base, P), pl.ds(xyz_off, 16)], cen_v)
        sg = [sg0, sg1]
        so = [so0, so1]
        w0 = [w_v[0, pl.ds(c * 16, 16)] for c in range(nc)]
        w1 = [w_v[1, pl.ds(c * 16, 16)] for c in range(nc)]
        w2 = [w_v[2, pl.ds(c * 16, 16)] for c in range(nc)]
        wb = [w_v[3, pl.ds(c * 16, 16)] for c in range(nc)]

        def gather_parts(t, buf, s):
            return [
                (tab_hbm.at[idx_v.at[pl.ds(t * RB + h * 128, 128)]],
                 rows_v.at[buf, pl.ds(h * 128, 128)], s)
                for h in range(nh)
            ]

        def start_gather(t, buf, s):
            for src, dst, sm in gather_parts(t, buf, s):
                pltpu.async_copy(src, dst, sm)

        def wait_gather(t, buf, s):
            for src, dst, sm in gather_parts(t, buf, s):
                pltpu.make_async_copy(src, dst, sm).wait()

        start_gather(0, 0, sg[0])

        def tt_body(tt, _):
            for b in range(2):
                t = tt * 2 + b

                @pl.when(t + 1 < nblk)
                def _():
                    start_gather(t + 1, 1 - b, sg[1 - b])

                wait_gather(t, b, sg[b])

                @pl.when(t >= 2)
                def _():
                    pltpu.make_async_copy(
                        out_v.at[b], agg_hbm.at[pl.ds(base, NB)],
                        so[b]).wait()

                def point(i, _):
                    cen = cen_v[t * NB + i, :]
                    accs = [jnp.zeros((16,), jnp.float32)
                            for _ in range(nc)]
                    for j in range(K):
                        r = i * K + j
                        diff = rows_v[b, r, pl.ds(xyz_off, 16)] - cen
                        rx = diff[0]
                        ry = diff[1]
                        rz = diff[2]
                        for c in range(nc):
                            t0 = (rx * w0[c] + ry * w1[c] + rz * w2[c]
                                  + wb[c])
                            w = _leaky(t0)
                            f = rows_v[b, r, pl.ds(feat_off + c * 16, 16)]
                            accs[c] = accs[c] + w * f
                    for c in range(nc):
                        out_v[b, i, pl.ds(c * 16, 16)] = accs[c]
                    return 0

                lax.fori_loop(0, NB, point, 0)
                pltpu.async_copy(
                    out_v.at[b], agg_hbm.at[pl.ds(base + t * NB, NB)],
                    so[b])
            return 0

        lax.fori_loop(0, nblk // 2, tt_body, 0)
        for b in range(2):
            pltpu.make_async_copy(
                out_v.at[b], agg_hbm.at[pl.ds(base, NB)], so[b]).wait()

    return k(table, idx, wp)


def _tc_linear(x, wT, b):
    M, C = x.shape
    O = wT.shape[1]
    BM = 2048

    def body(x_ref, w_ref, b_ref, o_ref):
        y = jnp.dot(x_ref[...], w_ref[...],
                    preferred_element_type=jnp.float32,
                    precision=lax.Precision.HIGHEST)
        y = y + b_ref[...]
        o_ref[...] = jnp.maximum(y, 0.1 * y)

    return pl.pallas_call(
        body,
        grid=(M // BM,),
        in_specs=[
            pl.BlockSpec((BM, C), lambda i: (i, 0)),
            pl.BlockSpec((C, O), lambda i: (0, 0)),
            pl.BlockSpec((1, O), lambda i: (0, 0)),
        ],
        out_specs=pl.BlockSpec((BM, O), lambda i: (i, 0)),
        out_shape=jax.ShapeDtypeStruct((M, O), jnp.float32),
    )(x, wT, b)


def _wpack(Wwn, bwn, pad_to=None):
    wp = jnp.concatenate([Wwn.T, bwn[None, :]], axis=0)
    if pad_to is not None and wp.shape[1] < pad_to:
        wp = jnp.pad(wp, ((0, 0), (0, pad_to - wp.shape[1])))
    return wp.astype(jnp.float32)


def _linpack(Wlin, blin, cin_pad=None, o_pad=128):
    wT = Wlin.T
    C, O = wT.shape
    if cin_pad is not None and C < cin_pad:
        wT = jnp.pad(wT, ((0, cin_pad - C), (0, 0)))
    if O < o_pad:
        wT = jnp.pad(wT, ((0, 0), (0, o_pad - O)))
        blin = jnp.pad(blin, (0, o_pad - O))
    return wT.astype(jnp.float32), blin[None, :].astype(jnp.float32)


def kernel(xyz, flow, corr, knn_indices,
           Wwn_c1, bwn_c1, Wlin_c1, blin_c1,
           Wwn_f1, bwn_f1, Wlin_f1, blin_f1,
           Wwn_f2, bwn_f2, Wlin_f2, blin_f2,
           Wwn_o, bwn_o, Wlin_o, blin_o):
    B, _, N = xyz.shape
    M = B * N
    f32 = jnp.float32
    corr = corr.astype(f32)
    flow = flow.astype(f32)

    xyzT = jnp.swapaxes(xyz, 1, 2).reshape(M, 3)
    flowT = jnp.swapaxes(flow, 1, 2).reshape(M, 3)
    z13 = jnp.zeros((M, 13), f32)
    xyz16 = jnp.concatenate([xyzT, z13], axis=1)
    tab0 = jnp.concatenate(
        [xyz16, flowT, z13], axis=1)
    offs = (jnp.arange(B, dtype=jnp.int32) * N)[:, None, None]
    idxg = knn_indices.astype(jnp.int32) + offs
    idx32 = idxg.reshape(M * 32)
    idx16 = idxg[:, :, :16].reshape(M * 16)
    corrT = jnp.swapaxes(corr, 1, 2).reshape(M, 128)

    agg_f1 = _sc_layer(tab0, idx32, _wpack(Wwn_f1, bwn_f1, pad_to=16),
                       M, C=0, K=32, NB=8, feat_off=16, R=32, Co=16)
    wT, bb = _linpack(Wlin_f1, blin_f1, cin_pad=16)
    ff1 = _tc_linear(agg_f1, wT, bb)[:, :32]

    tab_c = jnp.concatenate([corrT, xyz16], axis=1)
    agg_c1 = _sc_layer(tab_c, idx16, _wpack(Wwn_c1, bwn_c1),
                       M, C=128, K=16, NB=16, feat_off=0, R=144, Co=128)
    wT, bb = _linpack(Wlin_c1, blin_c1)
    cf = _tc_linear(agg_c1, wT, bb)

    tab_f = jnp.concatenate([ff1, xyz16], axis=1)
    agg_f2 = _sc_layer(tab_f, idx16, _wpack(Wwn_f2, bwn_f2),
                       M, C=32, K=16, NB=16, feat_off=0, R=48, Co=32)
    wT, bb = _linpack(Wlin_f2, blin_f2)
    ff2 = _tc_linear(agg_f2, wT, bb)[:, :16]

    tab_o = jnp.concatenate([cf, ff2, xyz16], axis=1)
    agg_o = _sc_layer(tab_o, idx16, _wpack(Wwn_o, bwn_o),
                      M, C=144, K=16, NB=16, feat_off=0, R=160, Co=144)
    wT, bb = _linpack(Wlin_o, blin_o)
    out = _tc_linear(agg_o, wT, bb)[:, :125]

    out = jnp.swapaxes(out.reshape(B, N, 125), 1, 2)
    return jnp.concatenate(
        [out, flow], axis=1)


# speedup vs baseline: 66.7491x; 1.0713x over previous
import functools

import jax
import jax.numpy as jnp
from jax import lax
from jax.experimental import pallas as pl
from jax.experimental.pallas import tpu as pltpu
from jax.experimental.pallas import tpu_sc as plsc

_NC = 2
_NS = 16
_NW = _NC * _NS
_LANES = 16


def _leaky(x):
    return jnp.maximum(x, 0.1 * x)


def _sc_layer(tables, idx, wp, M, K, NB, chunk_src, xyz_src, Co):
    P = M // _NW
    RB = NB * K
    nh = RB // 128
    assert RB % 128 == 0
    nblk = P // NB
    assert nblk % 2 == 0
    nc = Co // _LANES
    nt = len(tables)
    Rs = [t.shape[1] for t in tables]

    mesh = plsc.VectorSubcoreMesh(core_axis_name="c", subcore_axis_name="s")

    @functools.partial(
        pl.kernel,
        mesh=mesh,
        compiler_params=pltpu.CompilerParams(use_tc_tiling_on_sc=False),
        out_type=jax.ShapeDtypeStruct((M, Co), jnp.float32),
        scratch_types=[
            pltpu.VMEM((P * K,), jnp.int32),
            pltpu.VMEM((P, 16), jnp.float32),
            pltpu.VMEM((2, NB, Co), jnp.float32),
            pltpu.VMEM((4, Co), jnp.float32),
        ] + [pltpu.VMEM((2, RB, R), jnp.float32) for R in Rs] + [
            pltpu.SemaphoreType.DMA,
            pltpu.SemaphoreType.DMA,
            pltpu.SemaphoreType.DMA,
            pltpu.SemaphoreType.DMA,
        ],
    )
    def k(*refs):
        tabs_hbm = refs[:nt]
        idx_hbm, w_hbm, agg_hbm, idx_v, cen_v, out_v, w_v = refs[nt:nt + 7]
        rows_vs = refs[nt + 7:nt + 7 + nt]
        sg0, sg1, so0, so1 = refs[nt + 7 + nt:]
        wid = lax.axis_index("s") * _NC + lax.axis_index("c")
        base = wid * P
        pltpu.sync_copy(w_hbm, w_v)
        pltpu.sync_copy(idx_hbm.at[pl.ds(base * K, P * K)], idx_v)
        cti, cto = xyz_src
        pltpu.sync_copy(
            tabs_hbm[cti].at[pl.ds(base, P), pl.ds(cto, 16)], cen_v)
        sg = [sg0, sg1]
        so = [so0, so1]
        w0 = [w_v[0, pl.ds(c * 16, 16)] for c in range(nc)]
        w1 = [w_v[1, pl.ds(c * 16, 16)] for c in range(nc)]
        w2 = [w_v[2, pl.ds(c * 16, 16)] for c in range(nc)]
        wb = [w_v[3, pl.ds(c * 16, 16)] for c in range(nc)]

        def gather_parts(t, buf, s):
            return [
                (tabs_hbm[ti].at[idx_v.at[pl.ds(t * RB + h * 128, 128)]],
                 rows_vs[ti].at[buf, pl.ds(h * 128, 128)], s)
                for ti in range(nt)
                for h in range(nh)
            ]

        def start_gather(t, buf, s):
            for src, dst, sm in gather_parts(t, buf, s):
                pltpu.async_copy(src, dst, sm)

        def wait_gather(t, buf, s):
            for src, dst, sm in gather_parts(t, buf, s):
                pltpu.make_async_copy(src, dst, sm).wait()

        start_gather(0, 0, sg[0])

        def tt_body(tt, _):
            for b in range(2):
                t = tt * 2 + b

                @pl.when(t + 1 < nblk)
                def _():
                    start_gather(t + 1, 1 - b, sg[1 - b])

                wait_gather(t, b, sg[b])

                @pl.when(t >= 2)
                def _():
                    pltpu.make_async_copy(
                        out_v.at[b], agg_hbm.at[pl.ds(base, NB)],
                        so[b]).wait()

                xti, xto = xyz_src

                def point(i, _):
                    cen = cen_v[t * NB + i, :]
                    accs = [jnp.zeros((16,), jnp.float32)
                            for _ in range(nc)]
                    for j in range(K):
                        r = i * K + j
                        diff = rows_vs[xti][b, r, pl.ds(xto, 16)] - cen
                        rx = diff[0]
                        ry = diff[1]
                        rz = diff[2]
                        for c in range(nc):
                            t0 = (rx * w0[c] + ry * w1[c] + rz * w2[c]
                                  + wb[c])
                            w = _leaky(t0)
                            fti, fto = chunk_src[c]
                            f = rows_vs[fti][b, r, pl.ds(fto, 16)]
                            accs[c] = accs[c] + w * f
                    for c in range(nc):
                        out_v[b, i, pl.ds(c * 16, 16)] = accs[c]
                    return 0

                lax.fori_loop(0, NB, point, 0)
                pltpu.async_copy(
                    out_v.at[b], agg_hbm.at[pl.ds(base + t * NB, NB)],
                    so[b])
            return 0

        lax.fori_loop(0, nblk // 2, tt_body, 0)
        for b in range(2):
            pltpu.make_async_copy(
                out_v.at[b], agg_hbm.at[pl.ds(base, NB)], so[b]).wait()

    return k(*tables, idx, wp)


def _tc_linear(x, wT, b, x16=None):
    M, C = x.shape
    O = wT.shape[1]
    BM = 2048
    E = 16 if x16 is not None else 0

    def body(*refs):
        if E:
            x_ref, w_ref, b_ref, e_ref, o_ref = refs
        else:
            x_ref, w_ref, b_ref, o_ref = refs
        y = jnp.dot(x_ref[...], w_ref[...],
                    preferred_element_type=jnp.float32,
                    precision=lax.Precision.HIGHEST)
        y = y + b_ref[...]
        o_ref[:, :O] = jnp.maximum(y, 0.1 * y)
        if E:
            o_ref[:, O:] = e_ref[...]

    in_specs = [
        pl.BlockSpec((BM, C), lambda i: (i, 0)),
        pl.BlockSpec((C, O), lambda i: (0, 0)),
        pl.BlockSpec((1, O), lambda i: (0, 0)),
    ]
    args = [x, wT, b]
    if E:
        in_specs.append(pl.BlockSpec((BM, 16), lambda i: (i, 0)))
        args.append(x16)

    return pl.pallas_call(
        body,
        grid=(M // BM,),
        in_specs=in_specs,
        out_specs=pl.BlockSpec((BM, O + E), lambda i: (i, 0)),
        out_shape=jax.ShapeDtypeStruct((M, O + E), jnp.float32),
    )(*args)


def _tc_final(x, wT, b, B, N):
    BM = 2048

    def body(x_ref, w_ref, b_ref, o_ref):
        y = jnp.dot(x_ref[0], w_ref[...],
                    preferred_element_type=jnp.float32,
                    precision=lax.Precision.HIGHEST)
        y = y + b_ref[...]
        y = jnp.maximum(y, 0.1 * y)
        o_ref[0] = jnp.swapaxes(y, 0, 1)[:125, :]

    return pl.pallas_call(
        body,
        grid=(B, N // BM),
        in_specs=[
            pl.BlockSpec((1, BM, 144), lambda bi, i: (bi, i, 0)),
            pl.BlockSpec((144, 128), lambda bi, i: (0, 0)),
            pl.BlockSpec((1, 128), lambda bi, i: (0, 0)),
        ],
        out_specs=pl.BlockSpec((1, 125, BM), lambda bi, i: (bi, 0, i)),
        out_shape=jax.ShapeDtypeStruct((B, 125, N), jnp.float32),
    )(x, wT, b)


def _wpack(Wwn, bwn, pad_to=None):
    wp = jnp.concatenate([Wwn.T, bwn[None, :]], axis=0)
    if pad_to is not None and wp.shape[1] < pad_to:
        wp = jnp.pad(wp, ((0, 0), (0, pad_to - wp.shape[1])))
    return wp.astype(jnp.float32)


def _linpack(Wlin, blin, cin_pad=None, o_pad=None):
    wT = Wlin.T
    C, O = wT.shape
    if cin_pad is not None and C < cin_pad:
        wT = jnp.pad(wT, ((0, cin_pad - C), (0, 0)))
    if o_pad is not None and O < o_pad:
        wT = jnp.pad(wT, ((0, 0), (0, o_pad - O)))
        blin = jnp.pad(blin, (0, o_pad - O))
    return wT.astype(jnp.float32), blin[None, :].astype(jnp.float32)


def kernel(xyz, flow, corr, knn_indices,
           Wwn_c1, bwn_c1, Wlin_c1, blin_c1,
           Wwn_f1, bwn_f1, Wlin_f1, blin_f1,
           Wwn_f2, bwn_f2, Wlin_f2, blin_f2,
           Wwn_o, bwn_o, Wlin_o, blin_o):
    B, _, N = xyz.shape
    M = B * N
    f32 = jnp.float32
    corr = corr.astype(f32)
    flow = flow.astype(f32)

    xyzT = jnp.swapaxes(xyz, 1, 2).reshape(M, 3)
    flowT = jnp.swapaxes(flow, 1, 2).reshape(M, 3)
    z13 = jnp.zeros((M, 13), f32)
    xyz16 = jnp.concatenate([xyzT, z13], axis=1)
    tab0 = jnp.concatenate(
        [xyz16, flowT, z13], axis=1)
    offs = (jnp.arange(B, dtype=jnp.int32) * N)[:, None, None]
    idxg = knn_indices.astype(jnp.int32) + offs
    idx32 = idxg.reshape(M * 32)
    idx16 = idxg[:, :, :16].reshape(M * 16)
    corrT = jnp.swapaxes(corr, 1, 2).reshape(M, 128)

    agg_f1 = _sc_layer([tab0], idx32, _wpack(Wwn_f1, bwn_f1, pad_to=16),
                       M, K=32, NB=8,
                       chunk_src=[(0, 16)], xyz_src=(0, 0), Co=16)
    wT, bb = _linpack(Wlin_f1, blin_f1, cin_pad=16)
    tab_f = _tc_linear(agg_f1, wT, bb, x16=xyz16)

    agg_c1 = _sc_layer([corrT, xyz16], idx16, _wpack(Wwn_c1, bwn_c1),
                       M, K=16, NB=16,
                       chunk_src=[(0, c * 16) for c in range(8)],
                       xyz_src=(1, 0), Co=128)
    wT, bb = _linpack(Wlin_c1, blin_c1)
    cf = _tc_linear(agg_c1, wT, bb)

    agg_f2 = _sc_layer([tab_f], idx16, _wpack(Wwn_f2, bwn_f2),
                       M, K=16, NB=16,
                       chunk_src=[(0, 0), (0, 16)], xyz_src=(0, 32), Co=32)
    wT, bb = _linpack(Wlin_f2, blin_f2)
    ff2x = _tc_linear(agg_f2, wT, bb, x16=xyz16)

    agg_o = _sc_layer([cf, ff2x], idx16, _wpack(Wwn_o, bwn_o),
                      M, K=16, NB=16,
                      chunk_src=[(0, c * 16) for c in range(8)] + [(1, 0)],
                      xyz_src=(1, 16), Co=144)
    wT, bb = _linpack(Wlin_o, blin_o, o_pad=128)
    out125 = _tc_final(agg_o.reshape(B, N, 144), wT, bb, B, N)

    return jnp.concatenate(
        [out125, flow], axis=1)


# speedup vs baseline: 67.5511x; 1.0120x over previous
import functools

import jax
import jax.numpy as jnp
from jax import lax
from jax.experimental import pallas as pl
from jax.experimental.pallas import tpu as pltpu
from jax.experimental.pallas import tpu_sc as plsc

_NC = 2
_NS = 16
_NW = _NC * _NS
_LANES = 16


def _leaky(x):
    return jnp.maximum(x, 0.1 * x)


def _sc_layer(tables, idx, wp, M, K, NB, chunk_src, xyz_src, Co):
    P = M // _NW
    RB = NB * K
    nh = RB // 128
    assert RB % 128 == 0
    nblk = P // NB
    assert nblk % 2 == 0
    nc = Co // _LANES
    nt = len(tables)
    Rs = [t.shape[1] for t in tables]

    mesh = plsc.VectorSubcoreMesh(core_axis_name="c", subcore_axis_name="s")

    @functools.partial(
        pl.kernel,
        mesh=mesh,
        compiler_params=pltpu.CompilerParams(use_tc_tiling_on_sc=False),
        out_type=jax.ShapeDtypeStruct((M, Co), jnp.float32),
        scratch_types=[
            pltpu.VMEM((P * K,), jnp.int32),
            pltpu.VMEM((P, 16), jnp.float32),
            pltpu.VMEM((2, NB, Co), jnp.float32),
            pltpu.VMEM((4, Co), jnp.float32),
        ] + [pltpu.VMEM((2, RB, R), jnp.float32) for R in Rs] + [
            pltpu.SemaphoreType.DMA,
            pltpu.SemaphoreType.DMA,
            pltpu.SemaphoreType.DMA,
            pltpu.SemaphoreType.DMA,
        ],
    )
    def k(*refs):
        tabs_hbm = refs[:nt]
        idx_hbm, w_hbm, agg_hbm, idx_v, cen_v, out_v, w_v = refs[nt:nt + 7]
        rows_vs = refs[nt + 7:nt + 7 + nt]
        sg0, sg1, so0, so1 = refs[nt + 7 + nt:]
        wid = lax.axis_index("s") * _NC + lax.axis_index("c")
        base = wid * P
        pltpu.sync_copy(w_hbm, w_v)
        pltpu.sync_copy(idx_hbm.at[pl.ds(base * K, P * K)], idx_v)
        cti, cto = xyz_src
        pltpu.sync_copy(
            tabs_hbm[cti].at[pl.ds(base, P), pl.ds(cto, 16)], cen_v)
        sg = [sg0, sg1]
        so = [so0, so1]
        w0 = [w_v[0, pl.ds(c * 16, 16)] for c in range(nc)]
        w1 = [w_v[1, pl.ds(c * 16, 16)] for c in range(nc)]
        w2 = [w_v[2, pl.ds(c * 16, 16)] for c in range(nc)]
        wb = [w_v[3, pl.ds(c * 16, 16)] for c in range(nc)]

        def gather_parts(t, buf, s):
            return [
                (tabs_hbm[ti].at[idx_v.at[pl.ds(t * RB + h * 128, 128)]],
                 rows_vs[ti].at[buf, pl.ds(h * 128, 128)], s)
                for ti in range(nt)
                for h in range(nh)
            ]

        def start_gather(t, buf, s):
            for src, dst, sm in gather_parts(t, buf, s):
                pltpu.async_copy(src, dst, sm)

        def wait_gather(t, buf, s):
            for src, dst, sm in gather_parts(t, buf, s):
                pltpu.make_async_copy(src, dst, sm).wait()

        start_gather(0, 0, sg[0])

        def tt_body(tt, _):
            for b in range(2):
                t = tt * 2 + b

                @pl.when(t + 1 < nblk)
                def _():
                    start_gather(t + 1, 1 - b, sg[1 - b])

                wait_gather(t, b, sg[b])

                @pl.when(t >= 2)
                def _():
                    pltpu.make_async_copy(
                        out_v.at[b], agg_hbm.at[pl.ds(base, NB)],
                        so[b]).wait()

                xti, xto = xyz_src

                def point(i, _):
                    cen = cen_v[t * NB + i, :]
                    accs = [jnp.zeros((16,), jnp.float32)
                            for _ in range(nc)]
                    for j in range(K):
                        r = i * K + j
                        diff = rows_vs[xti][b, r, pl.ds(xto, 16)] - cen
                        rx = diff[0]
                        ry = diff[1]
                        rz = diff[2]
                        for c in range(nc):
                            t0 = (rx * w0[c] + ry * w1[c] + rz * w2[c]
                                  + wb[c])
                            w = _leaky(t0)
                            fti, fto = chunk_src[c]
                            f = rows_vs[fti][b, r, pl.ds(fto, 16)]
                            accs[c] = accs[c] + w * f
                    for c in range(nc):
                        out_v[b, i, pl.ds(c * 16, 16)] = accs[c]
                    return 0

                lax.fori_loop(0, NB, point, 0)
                pltpu.async_copy(
                    out_v.at[b], agg_hbm.at[pl.ds(base + t * NB, NB)],
                    so[b])
            return 0

        lax.fori_loop(0, nblk // 2, tt_body, 0)
        for b in range(2):
            pltpu.make_async_copy(
                out_v.at[b], agg_hbm.at[pl.ds(base, NB)], so[b]).wait()

    return k(*tables, idx, wp)


def _tc_linear(x, wT, b, x16=None):
    M, C = x.shape
    O = wT.shape[1]
    BM = 2048
    E = 16 if x16 is not None else 0

    def body(*refs):
        if E:
            x_ref, w_ref, b_ref, e_ref, o_ref = refs
        else:
            x_ref, w_ref, b_ref, o_ref = refs
        y = jnp.dot(x_ref[...], w_ref[...],
                    preferred_element_type=jnp.float32,
                    precision=lax.Precision.HIGHEST)
        y = y + b_ref[...]
        o_ref[:, :O] = jnp.maximum(y, 0.1 * y)
        if E:
            o_ref[:, O:] = e_ref[...]

    in_specs = [
        pl.BlockSpec((BM, C), lambda i: (i, 0)),
        pl.BlockSpec((C, O), lambda i: (0, 0)),
        pl.BlockSpec((1, O), lambda i: (0, 0)),
    ]
    args = [x, wT, b]
    if E:
        in_specs.append(pl.BlockSpec((BM, 16), lambda i: (i, 0)))
        args.append(x16)

    return pl.pallas_call(
        body,
        grid=(M // BM,),
        in_specs=in_specs,
        out_specs=pl.BlockSpec((BM, O + E), lambda i: (i, 0)),
        out_shape=jax.ShapeDtypeStruct((M, O + E), jnp.float32),
    )(*args)


def _tc_final(x, wT, b, flow, B, N):
    BM = 2048

    def body(x_ref, w_ref, b_ref, f_ref, o_ref):
        y = jnp.dot(x_ref[0], w_ref[...],
                    preferred_element_type=jnp.float32,
                    precision=lax.Precision.HIGHEST)
        y = y + b_ref[...]
        y = jnp.maximum(y, 0.1 * y)
        yt = jnp.swapaxes(y, 0, 1)
        o_ref[0] = jnp.concatenate([yt[:125, :], f_ref[0]], axis=0)

    return pl.pallas_call(
        body,
        grid=(B, N // BM),
        in_specs=[
            pl.BlockSpec((1, BM, 144), lambda bi, i: (bi, i, 0)),
            pl.BlockSpec((144, 128), lambda bi, i: (0, 0)),
            pl.BlockSpec((1, 128), lambda bi, i: (0, 0)),
            pl.BlockSpec((1, 3, BM), lambda bi, i: (bi, 0, i)),
        ],
        out_specs=pl.BlockSpec((1, 128, BM), lambda bi, i: (bi, 0, i)),
        out_shape=jax.ShapeDtypeStruct((B, 128, N), jnp.float32),
    )(x, wT, b, flow)


def _tc_transpose(corr, B, C, N):
    BM = 2048
    nb = N // BM

    def body(x_ref, o_ref):
        o_ref[...] = jnp.swapaxes(x_ref[0], 0, 1)

    return pl.pallas_call(
        body,
        grid=(B, nb),
        in_specs=[
            pl.BlockSpec((1, C, BM), lambda bi, i: (bi, 0, i)),
        ],
        out_specs=pl.BlockSpec((BM, C), lambda bi, i: (bi * nb + i, 0)),
        out_shape=jax.ShapeDtypeStruct((B * N, C), jnp.float32),
    )(corr)


def _wpack(Wwn, bwn, pad_to=None):
    wp = jnp.concatenate([Wwn.T, bwn[None, :]], axis=0)
    if pad_to is not None and wp.shape[1] < pad_to:
        wp = jnp.pad(wp, ((0, 0), (0, pad_to - wp.shape[1])))
    return wp.astype(jnp.float32)


def _linpack(Wlin, blin, cin_pad=None, o_pad=None):
    wT = Wlin.T
    C, O = wT.shape
    if cin_pad is not None and C < cin_pad:
        wT = jnp.pad(wT, ((0, cin_pad - C), (0, 0)))
    if o_pad is not None and O < o_pad:
        wT = jnp.pad(wT, ((0, 0), (0, o_pad - O)))
        blin = jnp.pad(blin, (0, o_pad - O))
    return wT.astype(jnp.float32), blin[None, :].astype(jnp.float32)


def kernel(xyz, flow, corr, knn_indices,
           Wwn_c1, bwn_c1, Wlin_c1, blin_c1,
           Wwn_f1, bwn_f1, Wlin_f1, blin_f1,
           Wwn_f2, bwn_f2, Wlin_f2, blin_f2,
           Wwn_o, bwn_o, Wlin_o, blin_o):
    B, _, N = xyz.shape
    M = B * N
    f32 = jnp.float32
    corr = corr.astype(f32)
    flow = flow.astype(f32)

    xyzT = jnp.swapaxes(xyz, 1, 2).reshape(M, 3)
    flowT = jnp.swapaxes(flow, 1, 2).reshape(M, 3)
    z13 = jnp.zeros((M, 13), f32)
    xyz16 = jnp.concatenate([xyzT, z13], axis=1)
    tab0 = jnp.concatenate(
        [xyz16, flowT, z13], axis=1)
    offs = (jnp.arange(B, dtype=jnp.int32) * N)[:, None, None]
    idxg = knn_indices.astype(jnp.int32) + offs
    idx32 = idxg.reshape(M * 32)
    idx16 = idxg[:, :, :16].reshape(M * 16)
    corrT = _tc_transpose(corr, B, 128, N)

    agg_f1 = _sc_layer([tab0], idx32, _wpack(Wwn_f1, bwn_f1, pad_to=16),
                       M, K=32, NB=8,
                       chunk_src=[(0, 16)], xyz_src=(0, 0), Co=16)
    wT, bb = _linpack(Wlin_f1, blin_f1, cin_pad=16)
    tab_f = _tc_linear(agg_f1, wT, bb, x16=xyz16)

    agg_c1 = _sc_layer([corrT, xyz16], idx16, _wpack(Wwn_c1, bwn_c1),
                       M, K=16, NB=16,
                       chunk_src=[(0, c * 16) for c in range(8)],
                       xyz_src=(1, 0), Co=128)
    wT, bb = _linpack(Wlin_c1, blin_c1)
    cf = _tc_linear(agg_c1, wT, bb)

    agg_f2 = _sc_layer([tab_f], idx16, _wpack(Wwn_f2, bwn_f2),
                       M, K=16, NB=16,
                       chunk_src=[(0, 0), (0, 16)], xyz_src=(0, 32), Co=32)
    wT, bb = _linpack(Wlin_f2, blin_f2)
    ff2x = _tc_linear(agg_f2, wT, bb, x16=xyz16)

    agg_o = _sc_layer([cf, ff2x], idx16, _wpack(Wwn_o, bwn_o),
                      M, K=16, NB=16,
                      chunk_src=[(0, c * 16) for c in range(8)] + [(1, 0)],
                      xyz_src=(1, 16), Co=144)
    wT, bb = _linpack(Wlin_o, blin_o, o_pad=128)
    return _tc_final(agg_o.reshape(B, N, 144), wT, bb, flow, B, N)
